# baseline (speedup 1.0000x reference)
import functools

import jax
import jax.numpy as jnp
from jax import lax
from jax.experimental import pallas as pl
from jax.experimental.pallas import tpu as pltpu
from jax.experimental.pallas import tpu_sc as plsc

_NC = 2
_NS = 16
_NW = _NC * _NS

_GATHER_CHUNK = 128


def _make_gather(n_rows, d):
    rows_per_w = n_rows // _NW
    n_chunks = rows_per_w // _GATHER_CHUNK
    mesh = plsc.VectorSubcoreMesh(core_axis_name="c", subcore_axis_name="s")

    @functools.partial(
        pl.kernel,
        mesh=mesh,
        out_type=jax.ShapeDtypeStruct((n_rows, d), jnp.float32),
        scratch_types=[
            pltpu.VMEM((_GATHER_CHUNK,), jnp.int32),
            pltpu.VMEM((_GATHER_CHUNK, d), jnp.float32),
            pltpu.SemaphoreType.DMA,
        ],
    )
    def gather_k(idx_hbm, table_hbm, out_hbm, idx_v, rows_v, sem):
        wid = lax.axis_index("s") * _NC + lax.axis_index("c")
        base = wid * rows_per_w
        for c in range(n_chunks):
            off = base + c * _GATHER_CHUNK
            pltpu.sync_copy(idx_hbm.at[pl.ds(off, _GATHER_CHUNK)], idx_v)
            pltpu.async_copy(table_hbm.at[idx_v], rows_v, sem).wait()
            pltpu.sync_copy(rows_v, out_hbm.at[pl.ds(off, _GATHER_CHUNK)])

    return gather_k


_N_SPLIT = 4


def _mlp_body(g_ref, w1_ref, b1_ref, w2_ref, b2_ref, w3_ref, b3_ref, out_ref):
    m = g_ref.shape[0]
    sub = m // _N_SPLIT
    for s in range(_N_SPLIT):
        sl = pl.ds(s * sub, sub)
        g = g_ref[sl, :].astype(jnp.bfloat16)
        h = jnp.dot(g, w1_ref[:], preferred_element_type=jnp.float32)
        h = jnp.maximum(h + b1_ref[:], 0.0).astype(jnp.bfloat16)
        h = jnp.dot(h, w2_ref[:], preferred_element_type=jnp.float32)
        h = jnp.maximum(h + b2_ref[:], 0.0)
        out_ref[sl, :] = jnp.sum(h * w3_ref[:], axis=1, keepdims=True) + b3_ref[:]


def _mlp(g, w1, b1, w2, b2, w3t, b3, block_m=2048):
    batch, k1 = g.shape
    hidden = w1.shape[1]
    return pl.pallas_call(
        _mlp_body,
        grid=(batch // block_m,),
        in_specs=[
            pl.BlockSpec((block_m, k1), lambda i: (i, 0)),
            pl.BlockSpec((k1, hidden), lambda i: (0, 0)),
            pl.BlockSpec((1, hidden), lambda i: (0, 0)),
            pl.BlockSpec((hidden, hidden), lambda i: (0, 0)),
            pl.BlockSpec((1, hidden), lambda i: (0, 0)),
            pl.BlockSpec((1, hidden), lambda i: (0, 0)),
            pl.BlockSpec((1, 1), lambda i: (0, 0)),
        ],
        out_specs=pl.BlockSpec((block_m, 1), lambda i: (i, 0)),
        out_shape=jax.ShapeDtypeStruct((batch, 1), jnp.float32),
    )(g, w1, b1, w2, b2, w3t, b3)


_N_CHUNKS = 2


def kernel(x, emb, W1, b1, W2, b2, W3, b3):
    batch = x.shape[0]
    d = emb.shape[1]
    hidden = W1.shape[1]
    idx_flat = x.astype(jnp.int32).reshape(-1)
    n_idx = idx_flat.shape[0]
    chunk_idx = n_idx // _N_CHUNKS
    gather_fn = _make_gather(chunk_idx, d)
    w1 = W1.astype(jnp.bfloat16)
    w2 = W2.astype(jnp.bfloat16)
    b1r = b1.reshape(1, hidden)
    b2r = b2.reshape(1, hidden)
    w3t = W3.reshape(1, hidden)
    b3r = b3.reshape(1, 1)
    gs = [
        gather_fn(lax.dynamic_slice_in_dim(idx_flat, c * chunk_idx, chunk_idx), emb)
        for c in range(_N_CHUNKS)
    ]
    outs = [
        _mlp(g.reshape(chunk_idx // 2, 2 * d), w1, b1r, w2, b2r, w3t, b3r)
        for g in gs
    ]
    return jnp.concatenate(outs, axis=0)


# speedup vs baseline: 1.0019x; 1.0019x over previous
import functools

import jax
import jax.numpy as jnp
from jax import lax
from jax.experimental import pallas as pl
from jax.experimental.pallas import tpu as pltpu
from jax.experimental.pallas import tpu_sc as plsc

_NC = 2
_NS = 16
_NW = _NC * _NS

_GATHER_CHUNK = 128


def _make_gather(n_rows, d):
    rows_per_w = n_rows // _NW
    n_chunks = rows_per_w // _GATHER_CHUNK
    mesh = plsc.VectorSubcoreMesh(core_axis_name="c", subcore_axis_name="s")

    @functools.partial(
        pl.kernel,
        mesh=mesh,
        out_type=jax.ShapeDtypeStruct((n_rows, d), jnp.float32),
        scratch_types=[
            pltpu.VMEM((n_chunks, _GATHER_CHUNK), jnp.int32),
            pltpu.VMEM((_GATHER_CHUNK, d), jnp.float32),
            pltpu.VMEM((_GATHER_CHUNK, d), jnp.float32),
            pltpu.SemaphoreType.DMA,
            pltpu.SemaphoreType.DMA,
        ],
    )
    def gather_k(idx_hbm, table_hbm, out_hbm, idx_v, rows_a, rows_b, gsem, ssem):
        wid = lax.axis_index("s") * _NC + lax.axis_index("c")
        base = wid * rows_per_w
        pltpu.sync_copy(idx_hbm.at[pl.ds(wid * n_chunks, n_chunks)], idx_v)
        bufs = (rows_a, rows_b)
        scats = [None, None]
        for c in range(n_chunks):
            buf = bufs[c % 2]
            if scats[c % 2] is not None:
                scats[c % 2].wait()
            pltpu.async_copy(table_hbm.at[idx_v.at[c]], buf, gsem).wait()
            scats[c % 2] = pltpu.async_copy(
                buf, out_hbm.at[pl.ds(base + c * _GATHER_CHUNK, _GATHER_CHUNK)],
                ssem)
        for s in scats:
            if s is not None:
                s.wait()

    return gather_k


_N_SPLIT = 4


def _mlp_body(g_ref, w1_ref, b1_ref, w2_ref, b2_ref, w3_ref, b3_ref, out_ref):
    m = g_ref.shape[0]
    sub = m // _N_SPLIT
    for s in range(_N_SPLIT):
        sl = pl.ds(s * sub, sub)
        g = g_ref[sl, :].astype(jnp.bfloat16)
        h = jnp.dot(g, w1_ref[:], preferred_element_type=jnp.float32)
        h = jnp.maximum(h + b1_ref[:], 0.0).astype(jnp.bfloat16)
        h = jnp.dot(h, w2_ref[:], preferred_element_type=jnp.float32)
        h = jnp.maximum(h + b2_ref[:], 0.0)
        out_ref[sl, :] = jnp.sum(h * w3_ref[:], axis=1, keepdims=True) + b3_ref[:]


def _mlp(g, w1, b1, w2, b2, w3t, b3, block_m=2048):
    batch, k1 = g.shape
    hidden = w1.shape[1]
    return pl.pallas_call(
        _mlp_body,
        grid=(batch // block_m,),
        in_specs=[
            pl.BlockSpec((block_m, k1), lambda i: (i, 0)),
            pl.BlockSpec((k1, hidden), lambda i: (0, 0)),
            pl.BlockSpec((1, hidden), lambda i: (0, 0)),
            pl.BlockSpec((hidden, hidden), lambda i: (0, 0)),
            pl.BlockSpec((1, hidden), lambda i: (0, 0)),
            pl.BlockSpec((1, hidden), lambda i: (0, 0)),
            pl.BlockSpec((1, 1), lambda i: (0, 0)),
        ],
        out_specs=pl.BlockSpec((block_m, 1), lambda i: (i, 0)),
        out_shape=jax.ShapeDtypeStruct((batch, 1), jnp.float32),
    )(g, w1, b1, w2, b2, w3t, b3)


_N_CHUNKS = 2


def kernel(x, emb, W1, b1, W2, b2, W3, b3):
    batch = x.shape[0]
    d = emb.shape[1]
    hidden = W1.shape[1]
    idx_flat = x.astype(jnp.int32).reshape(-1)
    n_idx = idx_flat.shape[0]
    chunk_idx = n_idx // _N_CHUNKS
    gather_fn = _make_gather(chunk_idx, d)
    w1 = W1.astype(jnp.bfloat16)
    w2 = W2.astype(jnp.bfloat16)
    b1r = b1.reshape(1, hidden)
    b2r = b2.reshape(1, hidden)
    w3t = W3.reshape(1, hidden)
    b3r = b3.reshape(1, 1)
    idx2d = idx_flat.reshape(-1, _GATHER_CHUNK)
    rows_per_slice = chunk_idx // _GATHER_CHUNK
    gs = [
        gather_fn(
            lax.dynamic_slice_in_dim(idx2d, c * rows_per_slice, rows_per_slice),
            emb,
        )
        for c in range(_N_CHUNKS)
    ]
    outs = [
        _mlp(g.reshape(chunk_idx // 2, 2 * d), w1, b1r, w2, b2r, w3t, b3r)
        for g in gs
    ]
    return jnp.concatenate(outs, axis=0)
